# baseline (speedup 1.0000x reference)
import functools

import jax
import jax.numpy as jnp
from jax import lax
from jax.experimental import pallas as pl
from jax.experimental.pallas import tpu as pltpu
from jax.experimental.pallas import tpu_sc as plsc

N = 50000
E = 800000
IN_DIM = 128
HID = 64
N_STEPS = 2
N_ETYPES = 4
NUM_LABELS = 16
QW = 16

LANES = 128
EDGE_ROWS = 6400
E_PAD = EDGE_ROWS * LANES
ROWS_PER_TILE = EDGE_ROWS // 16
CHUNK_ROWS = 40
N_CHUNKS = ROWS_PER_TILE // CHUNK_ROWS
GROUP = 5
BODIES = CHUNK_ROWS // (2 * GROUP)
N_PAD = 50176
ACC_PER_TILE = N_PAD // 16
STAGE_ROWS = ACC_PER_TILE // 2
Y_ROWS = 2 * N * 8

BN = 2000


def _sc_scatter_kernel(ym, g0, g1, g2, g3, dst_hbm, out,
                       gixv, dstv, rows, zbuf, stage, acc,
                       gsemA, gsemB, ssemA, ssemB):
    c = lax.axis_index("c")
    s = lax.axis_index("s")
    obase = s * ACC_PER_TILE

    def zrow(i, _):
        zbuf[i, pl.ds(0, QW)] = jnp.zeros((16,), jnp.float32)
        return 0

    lax.fori_loop(0, STAGE_ROWS, zrow, 0)

    def drain_scatters(sem):
        for _ in range(GROUP):
            pltpu.make_async_copy(rows.at[0], acc.at[dstv.at[0]], sem).wait()

    def edge_pass(g_hbm):
        ebase = s * ROWS_PER_TILE

        def outer(k, _):
            @pl.when(k > 0)
            def _():
                drain_scatters(ssemA)
                drain_scatters(ssemB)

            rb = ebase + k * CHUNK_ROWS
            pltpu.sync_copy(g_hbm.at[pl.ds(rb, CHUNK_ROWS)], gixv)
            pltpu.sync_copy(dst_hbm.at[pl.ds(rb, CHUNK_ROWS)], dstv)

            def body(k2, _):
                baseA = k2 * 2 * GROUP
                baseB = baseA + GROUP

                @pl.when(k2 > 0)
                def _():
                    drain_scatters(ssemA)

                ghA = [pltpu.async_copy(ym.at[gixv.at[baseA + b]],
                                        rows.at[b], gsemA)
                       for b in range(GROUP)]

                @pl.when(k2 > 0)
                def _():
                    drain_scatters(ssemB)

                ghB = [pltpu.async_copy(ym.at[gixv.at[baseB + b]],
                                        rows.at[GROUP + b], gsemB)
                       for b in range(GROUP)]
                for hh in ghA:
                    hh.wait()
                for b in range(GROUP):
                    pltpu.async_copy(rows.at[b], acc.at[dstv.at[baseA + b]],
                                     ssemA, add=True)
                for hh in ghB:
                    hh.wait()
                for b in range(GROUP):
                    pltpu.async_copy(rows.at[GROUP + b],
                                     acc.at[dstv.at[baseB + b]],
                                     ssemB, add=True)
                return 0

            return lax.fori_loop(0, BODIES, body, 0)

        lax.fori_loop(0, N_CHUNKS, outer, 0)
        drain_scatters(ssemA)
        drain_scatters(ssemB)

    def write_out(col):
        for k2 in range(2):
            rsl = pl.ds(obase + k2 * STAGE_ROWS, STAGE_ROWS)
            pltpu.sync_copy(acc.at[rsl], stage)
            pltpu.sync_copy(stage, out.at[rsl, pl.ds(col, QW)])

    for p in range(2):
        pltpu.sync_copy(zbuf, acc.at[pl.ds(obase, STAGE_ROWS)])
        pltpu.sync_copy(zbuf, acc.at[pl.ds(obase + STAGE_ROWS, STAGE_ROWS)])
        plsc.subcore_barrier()

        @pl.when(c == 0)
        def _():
            edge_pass(g0 if p == 0 else g1)

        @pl.when(c == 1)
        def _():
            edge_pass(g2 if p == 0 else g3)

        plsc.subcore_barrier()

        @pl.when(c == 0)
        def _():
            write_out(QW * p)

        @pl.when(c == 1)
        def _():
            write_out(QW * (2 + p))


@functools.lru_cache(maxsize=1)
def _sc_scatter_callable():
    mesh = plsc.VectorSubcoreMesh(core_axis_name="c", subcore_axis_name="s")
    return pl.kernel(
        _sc_scatter_kernel,
        out_type=jax.ShapeDtypeStruct((N_PAD, 128), jnp.float32),
        mesh=mesh,
        scratch_types=[
            pltpu.VMEM((CHUNK_ROWS, LANES), jnp.int32),
            pltpu.VMEM((CHUNK_ROWS, LANES), jnp.int32),
            pltpu.VMEM((2 * GROUP, LANES, QW), jnp.float32),
            pltpu.VMEM((STAGE_ROWS, QW), jnp.float32),
            pltpu.VMEM((STAGE_ROWS, QW), jnp.float32),
            pltpu.VMEM_SHARED((N_PAD, QW), jnp.float32),
            pltpu.SemaphoreType.DMA,
            pltpu.SemaphoreType.DMA,
            pltpu.SemaphoreType.DMA,
            pltpu.SemaphoreType.DMA,
        ],
        compiler_params=pltpu.CompilerParams(use_tc_tiling_on_sc=False),
    )


def _sc_scatter_add(ymr, g0, g1, g2, g3, dst2):
    return _sc_scatter_callable()(ymr, g0, g1, g2, g3, dst2)


def _linear_in(h, W_inT, b_in2):
    def body(h_ref, w_ref, b_ref, o_ref):
        o_ref[...] = jnp.dot(h_ref[...], w_ref[...],
                             preferred_element_type=jnp.float32) + b_ref[...]

    return pl.pallas_call(
        body,
        grid=(N // BN,),
        in_specs=[pl.BlockSpec((BN, IN_DIM), lambda i: (i, 0)),
                  pl.BlockSpec((IN_DIM, HID), lambda i: (0, 0)),
                  pl.BlockSpec((1, HID), lambda i: (0, 0))],
        out_specs=pl.BlockSpec((BN, HID), lambda i: (i, 0)),
        out_shape=jax.ShapeDtypeStruct((N, HID), jnp.float32),
    )(h, W_inT, b_in2)


def _msg_tables(x, W_stk0, W_stk1, b_stk0, b_stk1):
    def body(x_ref, w0_ref, w1_ref, b0_ref, b1_ref, ym_ref):
        xb = x_ref[...]
        ym_ref[0] = jnp.dot(xb, w0_ref[...],
                            preferred_element_type=jnp.float32) + b0_ref[...]
        ym_ref[1] = jnp.dot(xb, w1_ref[...],
                            preferred_element_type=jnp.float32) + b1_ref[...]

    return pl.pallas_call(
        body,
        grid=(N // BN,),
        in_specs=[pl.BlockSpec((BN, HID), lambda i: (i, 0)),
                  pl.BlockSpec((HID, 128), lambda i: (0, 0)),
                  pl.BlockSpec((HID, 128), lambda i: (0, 0)),
                  pl.BlockSpec((1, 128), lambda i: (0, 0)),
                  pl.BlockSpec((1, 128), lambda i: (0, 0))],
        out_specs=pl.BlockSpec((2, BN, 128), lambda i: (0, i, 0)),
        out_shape=jax.ShapeDtypeStruct((2, N, 128), jnp.float32),
    )(x, W_stk0, W_stk1, b_stk0, b_stk1)


def _gru(agg, x, W_ihT, b_ih2, W_hhT, b_hh2):
    def body(a_ref, x_ref, wi_ref, bi_ref, wh_ref, bh_ref, o_ref):
        xb = x_ref[...]
        gi = jnp.dot(a_ref[:, :HID], wi_ref[...],
                     preferred_element_type=jnp.float32) + bi_ref[...]
        gh = jnp.dot(xb, wh_ref[...],
                     preferred_element_type=jnp.float32) + bh_ref[...]
        r = jax.nn.sigmoid(gi[:, :HID] + gh[:, :HID])
        z = jax.nn.sigmoid(gi[:, HID:2 * HID] + gh[:, HID:2 * HID])
        n = jnp.tanh(gi[:, 2 * HID:] + r * gh[:, 2 * HID:])
        o_ref[...] = (1.0 - z) * n + z * xb

    return pl.pallas_call(
        body,
        grid=(N // BN,),
        in_specs=[pl.BlockSpec((BN, 128), lambda i: (i, 0)),
                  pl.BlockSpec((BN, HID), lambda i: (i, 0)),
                  pl.BlockSpec((HID, 3 * HID), lambda i: (0, 0)),
                  pl.BlockSpec((1, 3 * HID), lambda i: (0, 0)),
                  pl.BlockSpec((HID, 3 * HID), lambda i: (0, 0)),
                  pl.BlockSpec((1, 3 * HID), lambda i: (0, 0))],
        out_specs=pl.BlockSpec((BN, HID), lambda i: (i, 0)),
        out_shape=jax.ShapeDtypeStruct((N, HID), jnp.float32),
    )(agg, x, W_ihT, b_ih2, W_hhT, b_hh2)


def _pool_mlp(x, W1T, b1_2, W2T, b2_2):
    nblk = N // BN

    def body(x_ref, w1_ref, b1_ref, w2_ref, b2_ref, o_ref, acc_ref):
        i = pl.program_id(0)

        @pl.when(i == 0)
        def _():
            acc_ref[...] = jnp.zeros_like(acc_ref)

        acc_ref[...] += jnp.sum(x_ref[...], axis=0, keepdims=True)

        @pl.when(i == nblk - 1)
        def _():
            hg = acc_ref[...] * (1.0 / N)
            t = jnp.maximum(
                jnp.dot(hg, w1_ref[...],
                        preferred_element_type=jnp.float32) + b1_ref[...],
                0.0)
            o_ref[...] = jnp.dot(t, w2_ref[...],
                                 preferred_element_type=jnp.float32) + b2_ref[...]

    return pl.pallas_call(
        body,
        grid=(nblk,),
        in_specs=[pl.BlockSpec((BN, HID), lambda i: (i, 0)),
                  pl.BlockSpec((HID, HID // 2), lambda i: (0, 0)),
                  pl.BlockSpec((1, HID // 2), lambda i: (0, 0)),
                  pl.BlockSpec((HID // 2, NUM_LABELS), lambda i: (0, 0)),
                  pl.BlockSpec((1, NUM_LABELS), lambda i: (0, 0))],
        out_specs=pl.BlockSpec((1, NUM_LABELS), lambda i: (0, 0)),
        out_shape=jax.ShapeDtypeStruct((1, NUM_LABELS), jnp.float32),
        scratch_shapes=[pltpu.VMEM((1, HID), jnp.float32)],
    )(x, W1T, b1_2, W2T, b2_2)


def kernel(h, edge_index, etype, W_in, b_in, W_e, b_e, W_ih, b_ih, W_hh, b_hh,
           W1, b1, W2, b2):
    src = edge_index[0].astype(jnp.int32)
    dst = edge_index[1].astype(jnp.int32)
    et = etype.astype(jnp.int32)

    def gq(q):
        hh, v = divmod(q, 2)
        g = hh * (8 * N) + 8 * src + 2 * et + v
        return jnp.concatenate(
            [g, jnp.zeros((E_PAD - E,), jnp.int32)]).reshape(EDGE_ROWS, LANES)

    g0, g1, g2, g3 = gq(0), gq(1), gq(2), gq(3)
    pad_dst = N + jnp.arange(E_PAD - E, dtype=jnp.int32) % (N_PAD - N)
    dst2 = jnp.concatenate([dst, pad_dst]).reshape(EDGE_ROWS, LANES)

    W_inT = W_in.T
    b_in2 = b_in.reshape(1, HID)
    WeT = jnp.transpose(W_e, (0, 2, 1))
    HH = HID // 2
    W_stk0 = jnp.concatenate([WeT[i][:, :HH] for i in range(N_ETYPES)],
                             axis=1)
    W_stk1 = jnp.concatenate([WeT[i][:, HH:] for i in range(N_ETYPES)],
                             axis=1)
    b_stk0 = jnp.concatenate([b_e[i][:HH] for i in range(N_ETYPES)]
                             ).reshape(1, 128)
    b_stk1 = jnp.concatenate([b_e[i][HH:] for i in range(N_ETYPES)]
                             ).reshape(1, 128)
    W_ihT = W_ih.T
    b_ih2 = b_ih.reshape(1, 3 * HID)
    W_hhT = W_hh.T
    b_hh2 = b_hh.reshape(1, 3 * HID)
    W1T = W1.T
    b1_2 = b1.reshape(1, HID // 2)
    W2T = W2.T
    b2_2 = b2.reshape(1, NUM_LABELS)

    x = _linear_in(h, W_inT, b_in2)

    def step(_, xc):
        ym = _msg_tables(xc, W_stk0, W_stk1, b_stk0, b_stk1)
        ymr = ym.reshape(Y_ROWS, QW)
        agg = _sc_scatter_add(ymr, g0, g1, g2, g3, dst2)
        return _gru(agg, xc, W_ihT, b_ih2, W_hhT, b_hh2)

    x = lax.fori_loop(0, N_STEPS, step, x)
    return _pool_mlp(x, W1T, b1_2, W2T, b2_2)


# speedup vs baseline: 1.6578x; 1.6578x over previous
import functools

import jax
import jax.numpy as jnp
from jax import lax
from jax.experimental import pallas as pl
from jax.experimental.pallas import tpu as pltpu
from jax.experimental.pallas import tpu_sc as plsc

N = 50000
E = 800000
IN_DIM = 128
HID = 64
N_STEPS = 2
N_ETYPES = 4
NUM_LABELS = 16
QW = 16

LANES = 128
EDGE_ROWS = 6272
E_PAD = EDGE_ROWS * LANES
ROWS_PER_TILE = EDGE_ROWS // 16
CHUNK_ROWS = 56
N_CHUNKS = ROWS_PER_TILE // CHUNK_ROWS
GROUP = 4
BODIES = CHUNK_ROWS // (2 * GROUP)
N_PAD = 50176
ACC_PER_TILE = N_PAD // 16
STAGE_ROWS = ACC_PER_TILE // 2
Y_ROWS = 2 * N * 8

BN = 2000


def _sc_scatter_kernel(ym, g0, g1, g2, g3, dst_hbm, out,
                       gixv, dstv, rows, zbuf, stage, acc,
                       gsemA, gsemB, ssemA, ssemB):
    c = lax.axis_index("c")
    s = lax.axis_index("s")
    obase = s * ACC_PER_TILE

    def zrow(i, _):
        zbuf[i, pl.ds(0, QW)] = jnp.zeros((16,), jnp.float32)
        return 0

    lax.fori_loop(0, STAGE_ROWS, zrow, 0)

    def drain_scatters(sem):
        for _ in range(GROUP):
            pltpu.make_async_copy(rows.at[0], acc.at[dstv.at[0]], sem).wait()

    def edge_pass(g_hbm):
        ebase = s * ROWS_PER_TILE

        def outer(k, _):
            @pl.when(k > 0)
            def _():
                drain_scatters(ssemA)
                drain_scatters(ssemB)

            rb = ebase + k * CHUNK_ROWS
            pltpu.sync_copy(g_hbm.at[pl.ds(rb, CHUNK_ROWS)], gixv)
            pltpu.sync_copy(dst_hbm.at[pl.ds(rb, CHUNK_ROWS)], dstv)

            def body(k2, _):
                baseA = k2 * 2 * GROUP
                baseB = baseA + GROUP

                @pl.when(k2 > 0)
                def _():
                    drain_scatters(ssemA)

                ghA = [pltpu.async_copy(ym.at[gixv.at[baseA + b]],
                                        rows.at[b], gsemA)
                       for b in range(GROUP)]

                @pl.when(k2 > 0)
                def _():
                    drain_scatters(ssemB)

                ghB = [pltpu.async_copy(ym.at[gixv.at[baseB + b]],
                                        rows.at[GROUP + b], gsemB)
                       for b in range(GROUP)]
                for hh in ghA:
                    hh.wait()
                for b in range(GROUP):
                    pltpu.async_copy(rows.at[b], acc.at[dstv.at[baseA + b]],
                                     ssemA, add=True)
                for hh in ghB:
                    hh.wait()
                for b in range(GROUP):
                    pltpu.async_copy(rows.at[GROUP + b],
                                     acc.at[dstv.at[baseB + b]],
                                     ssemB, add=True)
                return 0

            return lax.fori_loop(0, BODIES, body, 0)

        lax.fori_loop(0, N_CHUNKS, outer, 0)
        drain_scatters(ssemA)
        drain_scatters(ssemB)

    def write_out(col):
        for k2 in range(2):
            rsl = pl.ds(obase + k2 * STAGE_ROWS, STAGE_ROWS)
            pltpu.sync_copy(acc.at[rsl], stage)
            pltpu.sync_copy(stage, out.at[rsl, pl.ds(col, QW)])

    for p in range(2):
        pltpu.sync_copy(zbuf, acc.at[pl.ds(obase, STAGE_ROWS)])
        pltpu.sync_copy(zbuf, acc.at[pl.ds(obase + STAGE_ROWS, STAGE_ROWS)])
        plsc.subcore_barrier()

        @pl.when(c == 0)
        def _():
            edge_pass(g0 if p == 0 else g1)

        @pl.when(c == 1)
        def _():
            edge_pass(g2 if p == 0 else g3)

        plsc.subcore_barrier()

        @pl.when(c == 0)
        def _():
            write_out(QW * p)

        @pl.when(c == 1)
        def _():
            write_out(QW * (2 + p))


@functools.lru_cache(maxsize=1)
def _sc_scatter_callable():
    mesh = plsc.VectorSubcoreMesh(core_axis_name="c", subcore_axis_name="s")
    return pl.kernel(
        _sc_scatter_kernel,
        out_type=jax.ShapeDtypeStruct((N_PAD, 128), jnp.float32),
        mesh=mesh,
        scratch_types=[
            pltpu.VMEM((CHUNK_ROWS, LANES), jnp.int32),
            pltpu.VMEM((CHUNK_ROWS, LANES), jnp.int32),
            pltpu.VMEM((2 * GROUP, LANES, QW), jnp.float32),
            pltpu.VMEM((STAGE_ROWS, QW), jnp.float32),
            pltpu.VMEM((STAGE_ROWS, QW), jnp.float32),
            pltpu.VMEM_SHARED((N_PAD, QW), jnp.float32),
            pltpu.SemaphoreType.DMA,
            pltpu.SemaphoreType.DMA,
            pltpu.SemaphoreType.DMA,
            pltpu.SemaphoreType.DMA,
        ],
        compiler_params=pltpu.CompilerParams(use_tc_tiling_on_sc=False),
    )


def _sc_scatter_add(ymr, g0, g1, g2, g3, dst2):
    return _sc_scatter_callable()(ymr, g0, g1, g2, g3, dst2)


def _linear_in(h, W_inT, b_in2):
    def body(h_ref, w_ref, b_ref, o_ref):
        o_ref[...] = jnp.dot(h_ref[...], w_ref[...],
                             preferred_element_type=jnp.float32) + b_ref[...]

    return pl.pallas_call(
        body,
        grid=(N // BN,),
        in_specs=[pl.BlockSpec((BN, IN_DIM), lambda i: (i, 0)),
                  pl.BlockSpec((IN_DIM, HID), lambda i: (0, 0)),
                  pl.BlockSpec((1, HID), lambda i: (0, 0))],
        out_specs=pl.BlockSpec((BN, HID), lambda i: (i, 0)),
        out_shape=jax.ShapeDtypeStruct((N, HID), jnp.float32),
    )(h, W_inT, b_in2)


def _msg_tables(x, W_stk0, W_stk1, b_stk0, b_stk1):
    def body(x_ref, w0_ref, w1_ref, b0_ref, b1_ref, ym_ref):
        xb = x_ref[...]
        ym_ref[0] = jnp.dot(xb, w0_ref[...],
                            preferred_element_type=jnp.float32) + b0_ref[...]
        ym_ref[1] = jnp.dot(xb, w1_ref[...],
                            preferred_element_type=jnp.float32) + b1_ref[...]

    return pl.pallas_call(
        body,
        grid=(N // BN,),
        in_specs=[pl.BlockSpec((BN, HID), lambda i: (i, 0)),
                  pl.BlockSpec((HID, 128), lambda i: (0, 0)),
                  pl.BlockSpec((HID, 128), lambda i: (0, 0)),
                  pl.BlockSpec((1, 128), lambda i: (0, 0)),
                  pl.BlockSpec((1, 128), lambda i: (0, 0))],
        out_specs=pl.BlockSpec((2, BN, 128), lambda i: (0, i, 0)),
        out_shape=jax.ShapeDtypeStruct((2, N, 128), jnp.float32),
    )(x, W_stk0, W_stk1, b_stk0, b_stk1)


def _gru(agg, x, W_ihT, b_ih2, W_hhT, b_hh2):
    def body(a_ref, x_ref, wi_ref, bi_ref, wh_ref, bh_ref, o_ref):
        xb = x_ref[...]
        gi = jnp.dot(a_ref[:, :HID], wi_ref[...],
                     preferred_element_type=jnp.float32) + bi_ref[...]
        gh = jnp.dot(xb, wh_ref[...],
                     preferred_element_type=jnp.float32) + bh_ref[...]
        r = jax.nn.sigmoid(gi[:, :HID] + gh[:, :HID])
        z = jax.nn.sigmoid(gi[:, HID:2 * HID] + gh[:, HID:2 * HID])
        n = jnp.tanh(gi[:, 2 * HID:] + r * gh[:, 2 * HID:])
        o_ref[...] = (1.0 - z) * n + z * xb

    return pl.pallas_call(
        body,
        grid=(N // BN,),
        in_specs=[pl.BlockSpec((BN, 128), lambda i: (i, 0)),
                  pl.BlockSpec((BN, HID), lambda i: (i, 0)),
                  pl.BlockSpec((HID, 3 * HID), lambda i: (0, 0)),
                  pl.BlockSpec((1, 3 * HID), lambda i: (0, 0)),
                  pl.BlockSpec((HID, 3 * HID), lambda i: (0, 0)),
                  pl.BlockSpec((1, 3 * HID), lambda i: (0, 0))],
        out_specs=pl.BlockSpec((BN, HID), lambda i: (i, 0)),
        out_shape=jax.ShapeDtypeStruct((N, HID), jnp.float32),
    )(agg, x, W_ihT, b_ih2, W_hhT, b_hh2)


def _pool_mlp(x, W1T, b1_2, W2T, b2_2):
    nblk = N // BN

    def body(x_ref, w1_ref, b1_ref, w2_ref, b2_ref, o_ref, acc_ref):
        i = pl.program_id(0)

        @pl.when(i == 0)
        def _():
            acc_ref[...] = jnp.zeros_like(acc_ref)

        acc_ref[...] += jnp.sum(x_ref[...], axis=0, keepdims=True)

        @pl.when(i == nblk - 1)
        def _():
            hg = acc_ref[...] * (1.0 / N)
            t = jnp.maximum(
                jnp.dot(hg, w1_ref[...],
                        preferred_element_type=jnp.float32) + b1_ref[...],
                0.0)
            o_ref[...] = jnp.dot(t, w2_ref[...],
                                 preferred_element_type=jnp.float32) + b2_ref[...]

    return pl.pallas_call(
        body,
        grid=(nblk,),
        in_specs=[pl.BlockSpec((BN, HID), lambda i: (i, 0)),
                  pl.BlockSpec((HID, HID // 2), lambda i: (0, 0)),
                  pl.BlockSpec((1, HID // 2), lambda i: (0, 0)),
                  pl.BlockSpec((HID // 2, NUM_LABELS), lambda i: (0, 0)),
                  pl.BlockSpec((1, NUM_LABELS), lambda i: (0, 0))],
        out_specs=pl.BlockSpec((1, NUM_LABELS), lambda i: (0, 0)),
        out_shape=jax.ShapeDtypeStruct((1, NUM_LABELS), jnp.float32),
        scratch_shapes=[pltpu.VMEM((1, HID), jnp.float32)],
    )(x, W1T, b1_2, W2T, b2_2)


def kernel(h, edge_index, etype, W_in, b_in, W_e, b_e, W_ih, b_ih, W_hh, b_hh,
           W1, b1, W2, b2):
    src = edge_index[0].astype(jnp.int32)
    dst = edge_index[1].astype(jnp.int32)
    et = etype.astype(jnp.int32)

    def gq(q):
        hh, v = divmod(q, 2)
        g = hh * (8 * N) + 8 * src + 2 * et + v
        return jnp.concatenate(
            [g, jnp.zeros((E_PAD - E,), jnp.int32)]).reshape(EDGE_ROWS, LANES)

    g0, g1, g2, g3 = gq(0), gq(1), gq(2), gq(3)
    pad_dst = N + jnp.arange(E_PAD - E, dtype=jnp.int32) % (N_PAD - N)
    dst2 = jnp.concatenate([dst, pad_dst]).reshape(EDGE_ROWS, LANES)

    W_inT = W_in.T
    b_in2 = b_in.reshape(1, HID)
    WeT = jnp.transpose(W_e, (0, 2, 1))
    HH = HID // 2
    W_stk0 = jnp.concatenate([WeT[i][:, :HH] for i in range(N_ETYPES)],
                             axis=1)
    W_stk1 = jnp.concatenate([WeT[i][:, HH:] for i in range(N_ETYPES)],
                             axis=1)
    b_stk0 = jnp.concatenate([b_e[i][:HH] for i in range(N_ETYPES)]
                             ).reshape(1, 128)
    b_stk1 = jnp.concatenate([b_e[i][HH:] for i in range(N_ETYPES)]
                             ).reshape(1, 128)
    W_ihT = W_ih.T
    b_ih2 = b_ih.reshape(1, 3 * HID)
    W_hhT = W_hh.T
    b_hh2 = b_hh.reshape(1, 3 * HID)
    W1T = W1.T
    b1_2 = b1.reshape(1, HID // 2)
    W2T = W2.T
    b2_2 = b2.reshape(1, NUM_LABELS)

    x = _linear_in(h, W_inT, b_in2)

    def step(_, xc):
        ym = _msg_tables(xc, W_stk0, W_stk1, b_stk0, b_stk1)
        ymr = ym.reshape(Y_ROWS, QW)
        agg = _sc_scatter_add(ymr, g0, g1, g2, g3, dst2)
        return _gru(agg, xc, W_ihT, b_ih2, W_hhT, b_hh2)

    x = lax.fori_loop(0, N_STEPS, step, x)
    return _pool_mlp(x, W1T, b1_2, W2T, b2_2)


# speedup vs baseline: 1.6782x; 1.0124x over previous
import functools

import jax
import jax.numpy as jnp
from jax import lax
from jax.experimental import pallas as pl
from jax.experimental.pallas import tpu as pltpu
from jax.experimental.pallas import tpu_sc as plsc

N = 50000
E = 800000
IN_DIM = 128
HID = 64
N_STEPS = 2
N_ETYPES = 4
NUM_LABELS = 16
QW = 16

LANES = 128
EDGE_ROWS = 6272
E_PAD = EDGE_ROWS * LANES
ROWS_PER_TILE = EDGE_ROWS // 16
CHUNK_ROWS = 56
N_CHUNKS = ROWS_PER_TILE // CHUNK_ROWS
GROUP = 4
BODIES = CHUNK_ROWS // (2 * GROUP)
N_PAD = 50176
ACC_PER_TILE = N_PAD // 16
STAGE_ROWS = ACC_PER_TILE // 2
Y_ROWS = 2 * N * 8

BN = 2000


def _sc_scatter_kernel(ym, g0, g1, g2, g3, dst_hbm, out,
                       gixv, dstv, rows, zbuf, stage, acc,
                       gsemA, gsemB, ssemA, ssemB):
    c = lax.axis_index("c")
    s = lax.axis_index("s")
    obase = s * ACC_PER_TILE

    def zrow(i, _):
        zbuf[i, pl.ds(0, QW)] = jnp.zeros((16,), jnp.float32)
        return 0

    lax.fori_loop(0, STAGE_ROWS, zrow, 0)

    def drain_scatters(sem):
        for _ in range(GROUP):
            pltpu.make_async_copy(rows.at[0], acc.at[dstv.at[0]], sem).wait()

    def edge_pass(g_hbm):
        ebase = s * ROWS_PER_TILE

        def outer(k, _):
            @pl.when(k > 0)
            def _():
                drain_scatters(ssemA)
                drain_scatters(ssemB)

            rb = ebase + k * CHUNK_ROWS
            pltpu.sync_copy(g_hbm.at[pl.ds(rb, CHUNK_ROWS)], gixv)
            pltpu.sync_copy(dst_hbm.at[pl.ds(rb, CHUNK_ROWS)], dstv)

            def body(k2, _):
                baseA = k2 * 2 * GROUP
                baseB = baseA + GROUP

                @pl.when(k2 > 0)
                def _():
                    drain_scatters(ssemA)

                ghA = [pltpu.async_copy(ym.at[gixv.at[baseA + b]],
                                        rows.at[b], gsemA)
                       for b in range(GROUP)]

                @pl.when(k2 > 0)
                def _():
                    drain_scatters(ssemB)

                ghB = [pltpu.async_copy(ym.at[gixv.at[baseB + b]],
                                        rows.at[GROUP + b], gsemB)
                       for b in range(GROUP)]
                for hh in ghA:
                    hh.wait()
                for b in range(GROUP):
                    pltpu.async_copy(rows.at[b], acc.at[dstv.at[baseA + b]],
                                     ssemA, add=True)
                for hh in ghB:
                    hh.wait()
                for b in range(GROUP):
                    pltpu.async_copy(rows.at[GROUP + b],
                                     acc.at[dstv.at[baseB + b]],
                                     ssemB, add=True)
                return 0

            return lax.fori_loop(0, BODIES, body, 0)

        lax.fori_loop(0, N_CHUNKS, outer, 0)
        drain_scatters(ssemA)
        drain_scatters(ssemB)

    def write_out(col):
        rsl = pl.ds(obase, ACC_PER_TILE)
        pltpu.sync_copy(acc.at[rsl], out.at[rsl, pl.ds(col, QW)])

    for p in range(2):
        pltpu.sync_copy(zbuf, acc.at[pl.ds(obase, STAGE_ROWS)])
        pltpu.sync_copy(zbuf, acc.at[pl.ds(obase + STAGE_ROWS, STAGE_ROWS)])
        plsc.subcore_barrier()

        @pl.when(c == 0)
        def _():
            edge_pass(g0 if p == 0 else g1)

        @pl.when(c == 1)
        def _():
            edge_pass(g2 if p == 0 else g3)

        plsc.subcore_barrier()

        @pl.when(c == 0)
        def _():
            write_out(QW * p)

        @pl.when(c == 1)
        def _():
            write_out(QW * (2 + p))


@functools.lru_cache(maxsize=1)
def _sc_scatter_callable():
    mesh = plsc.VectorSubcoreMesh(core_axis_name="c", subcore_axis_name="s")
    return pl.kernel(
        _sc_scatter_kernel,
        out_type=jax.ShapeDtypeStruct((N_PAD, 128), jnp.float32),
        mesh=mesh,
        scratch_types=[
            pltpu.VMEM((CHUNK_ROWS, LANES), jnp.int32),
            pltpu.VMEM((CHUNK_ROWS, LANES), jnp.int32),
            pltpu.VMEM((2 * GROUP, LANES, QW), jnp.float32),
            pltpu.VMEM((STAGE_ROWS, QW), jnp.float32),
            pltpu.VMEM((STAGE_ROWS, QW), jnp.float32),
            pltpu.VMEM_SHARED((N_PAD, QW), jnp.float32),
            pltpu.SemaphoreType.DMA,
            pltpu.SemaphoreType.DMA,
            pltpu.SemaphoreType.DMA,
            pltpu.SemaphoreType.DMA,
        ],
        compiler_params=pltpu.CompilerParams(use_tc_tiling_on_sc=False),
    )


def _sc_scatter_add(ymr, g0, g1, g2, g3, dst2):
    return _sc_scatter_callable()(ymr, g0, g1, g2, g3, dst2)


def _msg_out(xb, w0_ref, w1_ref, b0_ref, b1_ref, x_ref_out, ym_ref):
    x_ref_out[...] = xb
    ym_ref[0] = jnp.dot(xb, w0_ref[...],
                        preferred_element_type=jnp.float32) + b0_ref[...]
    ym_ref[1] = jnp.dot(xb, w1_ref[...],
                        preferred_element_type=jnp.float32) + b1_ref[...]


_MSG_SPECS = [pl.BlockSpec((HID, 128), lambda i: (0, 0)),
              pl.BlockSpec((HID, 128), lambda i: (0, 0)),
              pl.BlockSpec((1, 128), lambda i: (0, 0)),
              pl.BlockSpec((1, 128), lambda i: (0, 0))]
_MSG_OUT_SPECS = [pl.BlockSpec((BN, HID), lambda i: (i, 0)),
                  pl.BlockSpec((2, BN, 128), lambda i: (0, i, 0))]
_MSG_OUT_SHAPES = [jax.ShapeDtypeStruct((N, HID), jnp.float32),
                   jax.ShapeDtypeStruct((2, N, 128), jnp.float32)]


def _lin_msg(h, W_inT, b_in2, W_stk0, W_stk1, b_stk0, b_stk1):
    def body(h_ref, w_ref, b_ref, w0_ref, w1_ref, b0_ref, b1_ref,
             x_ref_out, ym_ref):
        xb = jnp.dot(h_ref[...], w_ref[...],
                     preferred_element_type=jnp.float32) + b_ref[...]
        _msg_out(xb, w0_ref, w1_ref, b0_ref, b1_ref, x_ref_out, ym_ref)

    return pl.pallas_call(
        body,
        grid=(N // BN,),
        in_specs=[pl.BlockSpec((BN, IN_DIM), lambda i: (i, 0)),
                  pl.BlockSpec((IN_DIM, HID), lambda i: (0, 0)),
                  pl.BlockSpec((1, HID), lambda i: (0, 0))] + _MSG_SPECS,
        out_specs=_MSG_OUT_SPECS,
        out_shape=_MSG_OUT_SHAPES,
    )(h, W_inT, b_in2, W_stk0, W_stk1, b_stk0, b_stk1)


def _gru_msg(agg, x, W_ihT, b_ih2, W_hhT, b_hh2,
             W_stk0, W_stk1, b_stk0, b_stk1):
    def body(a_ref, x_ref, wi_ref, bi_ref, wh_ref, bh_ref,
             w0_ref, w1_ref, b0_ref, b1_ref, x_ref_out, ym_ref):
        xb = x_ref[...]
        gi = jnp.dot(a_ref[:, :HID], wi_ref[...],
                     preferred_element_type=jnp.float32) + bi_ref[...]
        gh = jnp.dot(xb, wh_ref[...],
                     preferred_element_type=jnp.float32) + bh_ref[...]
        r = jax.nn.sigmoid(gi[:, :HID] + gh[:, :HID])
        z = jax.nn.sigmoid(gi[:, HID:2 * HID] + gh[:, HID:2 * HID])
        n = jnp.tanh(gi[:, 2 * HID:] + r * gh[:, 2 * HID:])
        xn = (1.0 - z) * n + z * xb
        _msg_out(xn, w0_ref, w1_ref, b0_ref, b1_ref, x_ref_out, ym_ref)

    return pl.pallas_call(
        body,
        grid=(N // BN,),
        in_specs=[pl.BlockSpec((BN, 128), lambda i: (i, 0)),
                  pl.BlockSpec((BN, HID), lambda i: (i, 0)),
                  pl.BlockSpec((HID, 3 * HID), lambda i: (0, 0)),
                  pl.BlockSpec((1, 3 * HID), lambda i: (0, 0)),
                  pl.BlockSpec((HID, 3 * HID), lambda i: (0, 0)),
                  pl.BlockSpec((1, 3 * HID), lambda i: (0, 0))] + _MSG_SPECS,
        out_specs=_MSG_OUT_SPECS,
        out_shape=_MSG_OUT_SHAPES,
    )(agg, x, W_ihT, b_ih2, W_hhT, b_hh2, W_stk0, W_stk1, b_stk0, b_stk1)


def _pool_mlp(x, W1T, b1_2, W2T, b2_2):
    nblk = N // BN

    def body(x_ref, w1_ref, b1_ref, w2_ref, b2_ref, o_ref, acc_ref):
        i = pl.program_id(0)

        @pl.when(i == 0)
        def _():
            acc_ref[...] = jnp.zeros_like(acc_ref)

        acc_ref[...] += jnp.sum(x_ref[...], axis=0, keepdims=True)

        @pl.when(i == nblk - 1)
        def _():
            hg = acc_ref[...] * (1.0 / N)
            t = jnp.maximum(
                jnp.dot(hg, w1_ref[...],
                        preferred_element_type=jnp.float32) + b1_ref[...],
                0.0)
            o_ref[...] = jnp.dot(t, w2_ref[...],
                                 preferred_element_type=jnp.float32) + b2_ref[...]

    return pl.pallas_call(
        body,
        grid=(nblk,),
        in_specs=[pl.BlockSpec((BN, HID), lambda i: (i, 0)),
                  pl.BlockSpec((HID, HID // 2), lambda i: (0, 0)),
                  pl.BlockSpec((1, HID // 2), lambda i: (0, 0)),
                  pl.BlockSpec((HID // 2, NUM_LABELS), lambda i: (0, 0)),
                  pl.BlockSpec((1, NUM_LABELS), lambda i: (0, 0))],
        out_specs=pl.BlockSpec((1, NUM_LABELS), lambda i: (0, 0)),
        out_shape=jax.ShapeDtypeStruct((1, NUM_LABELS), jnp.float32),
        scratch_shapes=[pltpu.VMEM((1, HID), jnp.float32)],
    )(x, W1T, b1_2, W2T, b2_2)


def kernel(h, edge_index, etype, W_in, b_in, W_e, b_e, W_ih, b_ih, W_hh, b_hh,
           W1, b1, W2, b2):
    src = edge_index[0].astype(jnp.int32)
    dst = edge_index[1].astype(jnp.int32)
    et = etype.astype(jnp.int32)

    def gq(q):
        hh, v = divmod(q, 2)
        g = hh * (8 * N) + 8 * src + 2 * et + v
        return jnp.concatenate(
            [g, jnp.zeros((E_PAD - E,), jnp.int32)]).reshape(EDGE_ROWS, LANES)

    g0, g1, g2, g3 = gq(0), gq(1), gq(2), gq(3)
    pad_dst = N + jnp.arange(E_PAD - E, dtype=jnp.int32) % (N_PAD - N)
    dst2 = jnp.concatenate([dst, pad_dst]).reshape(EDGE_ROWS, LANES)

    W_inT = W_in.T
    b_in2 = b_in.reshape(1, HID)
    WeT = jnp.transpose(W_e, (0, 2, 1))
    HH = HID // 2
    W_stk0 = jnp.concatenate([WeT[i][:, :HH] for i in range(N_ETYPES)],
                             axis=1)
    W_stk1 = jnp.concatenate([WeT[i][:, HH:] for i in range(N_ETYPES)],
                             axis=1)
    b_stk0 = jnp.concatenate([b_e[i][:HH] for i in range(N_ETYPES)]
                             ).reshape(1, 128)
    b_stk1 = jnp.concatenate([b_e[i][HH:] for i in range(N_ETYPES)]
                             ).reshape(1, 128)
    W_ihT = W_ih.T
    b_ih2 = b_ih.reshape(1, 3 * HID)
    W_hhT = W_hh.T
    b_hh2 = b_hh.reshape(1, 3 * HID)
    W1T = W1.T
    b1_2 = b1.reshape(1, HID // 2)
    W2T = W2.T
    b2_2 = b2.reshape(1, NUM_LABELS)

    x, ym = _lin_msg(h, W_inT, b_in2, W_stk0, W_stk1, b_stk0, b_stk1)

    def step(_, carry):
        xc, ymc = carry
        agg = _sc_scatter_add(ymc.reshape(Y_ROWS, QW), g0, g1, g2, g3, dst2)
        xn, ymn = _gru_msg(agg, xc, W_ihT, b_ih2, W_hhT, b_hh2,
                           W_stk0, W_stk1, b_stk0, b_stk1)
        return (xn, ymn)

    x, _ym = lax.fori_loop(0, N_STEPS, step, (x, ym))
    return _pool_mlp(x, W1T, b1_2, W2T, b2_2)


# speedup vs baseline: 1.6862x; 1.0048x over previous
import functools

import jax
import jax.numpy as jnp
from jax import lax
from jax.experimental import pallas as pl
from jax.experimental.pallas import tpu as pltpu
from jax.experimental.pallas import tpu_sc as plsc

N = 50000
E = 800000
IN_DIM = 128
HID = 64
N_STEPS = 2
N_ETYPES = 4
NUM_LABELS = 16
QW = 16

LANES = 128
EDGE_ROWS = 6272
E_PAD = EDGE_ROWS * LANES
ROWS_PER_TILE = EDGE_ROWS // 16
CHUNK_ROWS = 56
N_CHUNKS = ROWS_PER_TILE // CHUNK_ROWS
GROUP = 4
BODIES = CHUNK_ROWS // (2 * GROUP)
N_PAD = 50176
ACC_PER_TILE = N_PAD // 16
STAGE_ROWS = ACC_PER_TILE // 2
Y_ROWS = 2 * N * 8

BN = 2000


def _sc_scatter_kernel(ym, g0, g1, g2, g3, dst_hbm, out,
                       gixv, dstv, rows, zbuf, stage, acc,
                       gsemA, gsemB, ssemA, ssemB):
    c = lax.axis_index("c")
    s = lax.axis_index("s")
    obase = s * ACC_PER_TILE

    def zrow(i, _):
        zbuf[i, pl.ds(0, QW)] = jnp.zeros((16,), jnp.float32)
        return 0

    lax.fori_loop(0, STAGE_ROWS, zrow, 0)

    def drain_scatters(sem):
        for _ in range(GROUP):
            pltpu.make_async_copy(rows.at[0], acc.at[dstv.at[0]], sem).wait()

    def edge_pass(g_hbm):
        ebase = s * ROWS_PER_TILE

        def outer(k, _):
            @pl.when(k > 0)
            def _():
                drain_scatters(ssemA)
                drain_scatters(ssemB)

            rb = ebase + k * CHUNK_ROWS
            pltpu.sync_copy(g_hbm.at[pl.ds(rb, CHUNK_ROWS)], gixv)
            pltpu.sync_copy(dst_hbm.at[pl.ds(rb, CHUNK_ROWS)], dstv)

            def body(k2, _):
                baseA = k2 * 2 * GROUP
                baseB = baseA + GROUP

                @pl.when(k2 > 0)
                def _():
                    drain_scatters(ssemA)

                ghA = [pltpu.async_copy(ym.at[gixv.at[baseA + b]],
                                        rows.at[b], gsemA)
                       for b in range(GROUP)]

                @pl.when(k2 > 0)
                def _():
                    drain_scatters(ssemB)

                ghB = [pltpu.async_copy(ym.at[gixv.at[baseB + b]],
                                        rows.at[GROUP + b], gsemB)
                       for b in range(GROUP)]
                for hh in ghA:
                    hh.wait()
                for b in range(GROUP):
                    pltpu.async_copy(rows.at[b], acc.at[dstv.at[baseA + b]],
                                     ssemA, add=True)
                for hh in ghB:
                    hh.wait()
                for b in range(GROUP):
                    pltpu.async_copy(rows.at[GROUP + b],
                                     acc.at[dstv.at[baseB + b]],
                                     ssemB, add=True)
                return 0

            return lax.fori_loop(0, BODIES, body, 0)

        lax.fori_loop(0, N_CHUNKS, outer, 0)
        drain_scatters(ssemA)
        drain_scatters(ssemB)

    def write_out(col):
        rsl = pl.ds(obase, ACC_PER_TILE)
        pltpu.sync_copy(acc.at[rsl], out.at[rsl, pl.ds(col, QW)])

    for p in range(2):
        pltpu.sync_copy(zbuf, acc.at[pl.ds(obase, STAGE_ROWS)])
        pltpu.sync_copy(zbuf, acc.at[pl.ds(obase + STAGE_ROWS, STAGE_ROWS)])
        plsc.subcore_barrier()

        @pl.when(c == 0)
        def _():
            edge_pass(g0 if p == 0 else g1)

        @pl.when(c == 1)
        def _():
            edge_pass(g2 if p == 0 else g3)

        plsc.subcore_barrier()

        @pl.when(c == 0)
        def _():
            write_out(QW * p)

        @pl.when(c == 1)
        def _():
            write_out(QW * (2 + p))


@functools.lru_cache(maxsize=1)
def _sc_scatter_callable():
    mesh = plsc.VectorSubcoreMesh(core_axis_name="c", subcore_axis_name="s")
    return pl.kernel(
        _sc_scatter_kernel,
        out_type=jax.ShapeDtypeStruct((N_PAD, 128), jnp.float32),
        mesh=mesh,
        scratch_types=[
            pltpu.VMEM((CHUNK_ROWS, LANES), jnp.int32),
            pltpu.VMEM((CHUNK_ROWS, LANES), jnp.int32),
            pltpu.VMEM((2 * GROUP, LANES, QW), jnp.float32),
            pltpu.VMEM((STAGE_ROWS, QW), jnp.float32),
            pltpu.VMEM((STAGE_ROWS, QW), jnp.float32),
            pltpu.VMEM_SHARED((N_PAD, QW), jnp.float32),
            pltpu.SemaphoreType.DMA,
            pltpu.SemaphoreType.DMA,
            pltpu.SemaphoreType.DMA,
            pltpu.SemaphoreType.DMA,
        ],
        compiler_params=pltpu.CompilerParams(use_tc_tiling_on_sc=False),
    )


def _sc_scatter_add(ymr, g0, g1, g2, g3, dst2):
    return _sc_scatter_callable()(ymr, g0, g1, g2, g3, dst2)


def _msg_out(xb, w0_ref, w1_ref, b0_ref, b1_ref, x_ref_out, ym_ref):
    x_ref_out[...] = xb
    ym_ref[0] = jnp.dot(xb, w0_ref[...],
                        preferred_element_type=jnp.float32) + b0_ref[...]
    ym_ref[1] = jnp.dot(xb, w1_ref[...],
                        preferred_element_type=jnp.float32) + b1_ref[...]


_MSG_SPECS = [pl.BlockSpec((HID, 128), lambda i: (0, 0)),
              pl.BlockSpec((HID, 128), lambda i: (0, 0)),
              pl.BlockSpec((1, 128), lambda i: (0, 0)),
              pl.BlockSpec((1, 128), lambda i: (0, 0))]
_MSG_OUT_SPECS = [pl.BlockSpec((BN, HID), lambda i: (i, 0)),
                  pl.BlockSpec((2, BN, 128), lambda i: (0, i, 0))]
_MSG_OUT_SHAPES = [jax.ShapeDtypeStruct((N, HID), jnp.float32),
                   jax.ShapeDtypeStruct((2, N, 128), jnp.float32)]


def _lin_msg(h, W_inT, b_in2, W_stk0, W_stk1, b_stk0, b_stk1):
    def body(h_ref, w_ref, b_ref, w0_ref, w1_ref, b0_ref, b1_ref,
             x_ref_out, ym_ref):
        xb = jnp.dot(h_ref[...], w_ref[...],
                     preferred_element_type=jnp.float32) + b_ref[...]
        _msg_out(xb, w0_ref, w1_ref, b0_ref, b1_ref, x_ref_out, ym_ref)

    return pl.pallas_call(
        body,
        grid=(N // BN,),
        in_specs=[pl.BlockSpec((BN, IN_DIM), lambda i: (i, 0)),
                  pl.BlockSpec((IN_DIM, HID), lambda i: (0, 0)),
                  pl.BlockSpec((1, HID), lambda i: (0, 0))] + _MSG_SPECS,
        out_specs=_MSG_OUT_SPECS,
        out_shape=_MSG_OUT_SHAPES,
    )(h, W_inT, b_in2, W_stk0, W_stk1, b_stk0, b_stk1)


def _gru_msg(agg, x, W_ihT, b_ih2, W_hhT, b_hh2,
             W_stk0, W_stk1, b_stk0, b_stk1):
    def body(a_ref, x_ref, wi_ref, bi_ref, wh_ref, bh_ref,
             w0_ref, w1_ref, b0_ref, b1_ref, x_ref_out, ym_ref):
        xb = x_ref[...]
        gi = jnp.dot(a_ref[:, :HID], wi_ref[...],
                     preferred_element_type=jnp.float32) + bi_ref[...]
        gh = jnp.dot(xb, wh_ref[...],
                     preferred_element_type=jnp.float32) + bh_ref[...]
        r = jax.nn.sigmoid(gi[:, :HID] + gh[:, :HID])
        z = jax.nn.sigmoid(gi[:, HID:2 * HID] + gh[:, HID:2 * HID])
        n = jnp.tanh(gi[:, 2 * HID:] + r * gh[:, 2 * HID:])
        xn = (1.0 - z) * n + z * xb
        _msg_out(xn, w0_ref, w1_ref, b0_ref, b1_ref, x_ref_out, ym_ref)

    return pl.pallas_call(
        body,
        grid=(N // BN,),
        in_specs=[pl.BlockSpec((BN, 128), lambda i: (i, 0)),
                  pl.BlockSpec((BN, HID), lambda i: (i, 0)),
                  pl.BlockSpec((HID, 3 * HID), lambda i: (0, 0)),
                  pl.BlockSpec((1, 3 * HID), lambda i: (0, 0)),
                  pl.BlockSpec((HID, 3 * HID), lambda i: (0, 0)),
                  pl.BlockSpec((1, 3 * HID), lambda i: (0, 0))] + _MSG_SPECS,
        out_specs=_MSG_OUT_SPECS,
        out_shape=_MSG_OUT_SHAPES,
        input_output_aliases={1: 0},
    )(agg, x, W_ihT, b_ih2, W_hhT, b_hh2, W_stk0, W_stk1, b_stk0, b_stk1)


def _pool_mlp(x, W1T, b1_2, W2T, b2_2):
    nblk = N // BN

    def body(x_ref, w1_ref, b1_ref, w2_ref, b2_ref, o_ref, acc_ref):
        i = pl.program_id(0)

        @pl.when(i == 0)
        def _():
            acc_ref[...] = jnp.zeros_like(acc_ref)

        acc_ref[...] += jnp.sum(x_ref[...], axis=0, keepdims=True)

        @pl.when(i == nblk - 1)
        def _():
            hg = acc_ref[...] * (1.0 / N)
            t = jnp.maximum(
                jnp.dot(hg, w1_ref[...],
                        preferred_element_type=jnp.float32) + b1_ref[...],
                0.0)
            o_ref[...] = jnp.dot(t, w2_ref[...],
                                 preferred_element_type=jnp.float32) + b2_ref[...]

    return pl.pallas_call(
        body,
        grid=(nblk,),
        in_specs=[pl.BlockSpec((BN, HID), lambda i: (i, 0)),
                  pl.BlockSpec((HID, HID // 2), lambda i: (0, 0)),
                  pl.BlockSpec((1, HID // 2), lambda i: (0, 0)),
                  pl.BlockSpec((HID // 2, NUM_LABELS), lambda i: (0, 0)),
                  pl.BlockSpec((1, NUM_LABELS), lambda i: (0, 0))],
        out_specs=pl.BlockSpec((1, NUM_LABELS), lambda i: (0, 0)),
        out_shape=jax.ShapeDtypeStruct((1, NUM_LABELS), jnp.float32),
        scratch_shapes=[pltpu.VMEM((1, HID), jnp.float32)],
    )(x, W1T, b1_2, W2T, b2_2)


def kernel(h, edge_index, etype, W_in, b_in, W_e, b_e, W_ih, b_ih, W_hh, b_hh,
           W1, b1, W2, b2):
    src = edge_index[0].astype(jnp.int32)
    dst = edge_index[1].astype(jnp.int32)
    et = etype.astype(jnp.int32)

    def gq(q):
        hh, v = divmod(q, 2)
        g = hh * (8 * N) + 8 * src + 2 * et + v
        return jnp.concatenate(
            [g, jnp.zeros((E_PAD - E,), jnp.int32)]).reshape(EDGE_ROWS, LANES)

    g0, g1, g2, g3 = gq(0), gq(1), gq(2), gq(3)
    pad_dst = N + jnp.arange(E_PAD - E, dtype=jnp.int32) % (N_PAD - N)
    dst2 = jnp.concatenate([dst, pad_dst]).reshape(EDGE_ROWS, LANES)

    W_inT = W_in.T
    b_in2 = b_in.reshape(1, HID)
    WeT = jnp.transpose(W_e, (0, 2, 1))
    HH = HID // 2
    W_stk0 = jnp.concatenate([WeT[i][:, :HH] for i in range(N_ETYPES)],
                             axis=1)
    W_stk1 = jnp.concatenate([WeT[i][:, HH:] for i in range(N_ETYPES)],
                             axis=1)
    b_stk0 = jnp.concatenate([b_e[i][:HH] for i in range(N_ETYPES)]
                             ).reshape(1, 128)
    b_stk1 = jnp.concatenate([b_e[i][HH:] for i in range(N_ETYPES)]
                             ).reshape(1, 128)
    W_ihT = W_ih.T
    b_ih2 = b_ih.reshape(1, 3 * HID)
    W_hhT = W_hh.T
    b_hh2 = b_hh.reshape(1, 3 * HID)
    W1T = W1.T
    b1_2 = b1.reshape(1, HID // 2)
    W2T = W2.T
    b2_2 = b2.reshape(1, NUM_LABELS)

    x, ym = _lin_msg(h, W_inT, b_in2, W_stk0, W_stk1, b_stk0, b_stk1)

    def step(_, carry):
        xc, ymc = carry
        agg = _sc_scatter_add(ymc.reshape(Y_ROWS, QW), g0, g1, g2, g3, dst2)
        xn, ymn = _gru_msg(agg, xc, W_ihT, b_ih2, W_hhT, b_hh2,
                           W_stk0, W_stk1, b_stk0, b_stk1)
        return (xn, ymn)

    x, _ym = lax.fori_loop(0, N_STEPS, step, (x, ym))
    return _pool_mlp(x, W1T, b1_2, W2T, b2_2)


# speedup vs baseline: 1.7194x; 1.0197x over previous
import functools

import jax
import jax.numpy as jnp
from jax import lax
from jax.experimental import pallas as pl
from jax.experimental.pallas import tpu as pltpu
from jax.experimental.pallas import tpu_sc as plsc

N = 50000
E = 800000
IN_DIM = 128
HID = 64
N_STEPS = 2
N_ETYPES = 4
NUM_LABELS = 16
QW = 16

LANES = 128
EDGE_ROWS = 6272
E_PAD = EDGE_ROWS * LANES
ROWS_PER_TILE = EDGE_ROWS // 16
CHUNK_ROWS = 56
N_CHUNKS = ROWS_PER_TILE // CHUNK_ROWS
GROUP = 4
BODIES = CHUNK_ROWS // (2 * GROUP)
N_PAD = 50176
ACC_PER_TILE = N_PAD // 16
STAGE_ROWS = ACC_PER_TILE // 2
Y_ROWS = 2 * N * 4

BN = 2000


def _sc_scatter_kernel(ym, g0, g1, dst_hbm, out,
                       gixv, dstv, rows, zbuf, acc,
                       gsemA, gsemB, ssemA, ssemB):
    c = lax.axis_index("c")
    s = lax.axis_index("s")
    obase = s * ACC_PER_TILE

    def zrow(i, _):
        zbuf[i, pl.ds(0, 2 * QW)] = jnp.zeros((32,), jnp.bfloat16)
        return 0

    lax.fori_loop(0, STAGE_ROWS, zrow, 0)
    pltpu.sync_copy(zbuf, acc.at[pl.ds(obase, STAGE_ROWS)])
    pltpu.sync_copy(zbuf, acc.at[pl.ds(obase + STAGE_ROWS, STAGE_ROWS)])
    plsc.subcore_barrier()

    def drain_scatters(sem):
        for _ in range(GROUP):
            pltpu.make_async_copy(rows.at[0], acc.at[dstv.at[0]], sem).wait()

    def edge_pass(g_hbm):
        ebase = s * ROWS_PER_TILE

        def outer(k, _):
            @pl.when(k > 0)
            def _():
                drain_scatters(ssemA)
                drain_scatters(ssemB)

            rb = ebase + k * CHUNK_ROWS
            pltpu.sync_copy(g_hbm.at[pl.ds(rb, CHUNK_ROWS)], gixv)
            pltpu.sync_copy(dst_hbm.at[pl.ds(rb, CHUNK_ROWS)], dstv)

            def body(k2, _):
                baseA = k2 * 2 * GROUP
                baseB = baseA + GROUP

                @pl.when(k2 > 0)
                def _():
                    drain_scatters(ssemA)

                ghA = [pltpu.async_copy(ym.at[gixv.at[baseA + b]],
                                        rows.at[b], gsemA)
                       for b in range(GROUP)]

                @pl.when(k2 > 0)
                def _():
                    drain_scatters(ssemB)

                ghB = [pltpu.async_copy(ym.at[gixv.at[baseB + b]],
                                        rows.at[GROUP + b], gsemB)
                       for b in range(GROUP)]
                for hh in ghA:
                    hh.wait()
                for b in range(GROUP):
                    pltpu.async_copy(rows.at[b], acc.at[dstv.at[baseA + b]],
                                     ssemA, add=True)
                for hh in ghB:
                    hh.wait()
                for b in range(GROUP):
                    pltpu.async_copy(rows.at[GROUP + b],
                                     acc.at[dstv.at[baseB + b]],
                                     ssemB, add=True)
                return 0

            return lax.fori_loop(0, BODIES, body, 0)

        lax.fori_loop(0, N_CHUNKS, outer, 0)
        drain_scatters(ssemA)
        drain_scatters(ssemB)

    @pl.when(c == 0)
    def _():
        edge_pass(g0)

    @pl.when(c == 1)
    def _():
        edge_pass(g1)

    plsc.subcore_barrier()

    rsl = pl.ds(obase, ACC_PER_TILE)

    @pl.when(c == 0)
    def _():
        pltpu.sync_copy(acc.at[rsl], out.at[rsl, pl.ds(0, 2 * QW)])

    @pl.when(c == 1)
    def _():
        pltpu.sync_copy(acc.at[rsl], out.at[rsl, pl.ds(2 * QW, 2 * QW)])


@functools.lru_cache(maxsize=1)
def _sc_scatter_callable():
    mesh = plsc.VectorSubcoreMesh(core_axis_name="c", subcore_axis_name="s")
    return pl.kernel(
        _sc_scatter_kernel,
        out_type=jax.ShapeDtypeStruct((N_PAD, 128), jnp.bfloat16),
        mesh=mesh,
        scratch_types=[
            pltpu.VMEM((CHUNK_ROWS, LANES), jnp.int32),
            pltpu.VMEM((CHUNK_ROWS, LANES), jnp.int32),
            pltpu.VMEM((2 * GROUP, LANES, 2 * QW), jnp.bfloat16),
            pltpu.VMEM((STAGE_ROWS, 2 * QW), jnp.bfloat16),
            pltpu.VMEM_SHARED((N_PAD, 2 * QW), jnp.bfloat16),
            pltpu.SemaphoreType.DMA,
            pltpu.SemaphoreType.DMA,
            pltpu.SemaphoreType.DMA,
            pltpu.SemaphoreType.DMA,
        ],
        compiler_params=pltpu.CompilerParams(use_tc_tiling_on_sc=False),
    )


def _sc_scatter_add(ymr, g0, g1, dst2):
    return _sc_scatter_callable()(ymr, g0, g1, dst2)


def _msg_out(xb, w0_ref, w1_ref, b0_ref, b1_ref, x_ref_out, ym_ref):
    x_ref_out[...] = xb
    y0 = jnp.dot(xb, w0_ref[...],
                 preferred_element_type=jnp.float32) + b0_ref[...]
    y1 = jnp.dot(xb, w1_ref[...],
                 preferred_element_type=jnp.float32) + b1_ref[...]
    ym_ref[0] = y0.astype(jnp.bfloat16)
    ym_ref[1] = y1.astype(jnp.bfloat16)


_MSG_SPECS = [pl.BlockSpec((HID, 128), lambda i: (0, 0)),
              pl.BlockSpec((HID, 128), lambda i: (0, 0)),
              pl.BlockSpec((1, 128), lambda i: (0, 0)),
              pl.BlockSpec((1, 128), lambda i: (0, 0))]
_MSG_OUT_SPECS = [pl.BlockSpec((BN, HID), lambda i: (i, 0)),
                  pl.BlockSpec((2, BN, 128), lambda i: (0, i, 0))]
_MSG_OUT_SHAPES = [jax.ShapeDtypeStruct((N, HID), jnp.float32),
                   jax.ShapeDtypeStruct((2, N, 128), jnp.bfloat16)]


def _lin_msg(h, W_inT, b_in2, W_stk0, W_stk1, b_stk0, b_stk1):
    def body(h_ref, w_ref, b_ref, w0_ref, w1_ref, b0_ref, b1_ref,
             x_ref_out, ym_ref):
        xb = jnp.dot(h_ref[...], w_ref[...],
                     preferred_element_type=jnp.float32) + b_ref[...]
        _msg_out(xb, w0_ref, w1_ref, b0_ref, b1_ref, x_ref_out, ym_ref)

    return pl.pallas_call(
        body,
        grid=(N // BN,),
        in_specs=[pl.BlockSpec((BN, IN_DIM), lambda i: (i, 0)),
                  pl.BlockSpec((IN_DIM, HID), lambda i: (0, 0)),
                  pl.BlockSpec((1, HID), lambda i: (0, 0))] + _MSG_SPECS,
        out_specs=_MSG_OUT_SPECS,
        out_shape=_MSG_OUT_SHAPES,
    )(h, W_inT, b_in2, W_stk0, W_stk1, b_stk0, b_stk1)


def _gru_msg(agg, x, W_ihT, b_ih2, W_hhT, b_hh2,
             W_stk0, W_stk1, b_stk0, b_stk1):
    def body(a_ref, x_ref, wi_ref, bi_ref, wh_ref, bh_ref,
             w0_ref, w1_ref, b0_ref, b1_ref, x_ref_out, ym_ref):
        xb = x_ref[...]
        gi = jnp.dot(a_ref[:, :HID].astype(jnp.float32), wi_ref[...],
                     preferred_element_type=jnp.float32) + bi_ref[...]
        gh = jnp.dot(xb, wh_ref[...],
                     preferred_element_type=jnp.float32) + bh_ref[...]
        r = jax.nn.sigmoid(gi[:, :HID] + gh[:, :HID])
        z = jax.nn.sigmoid(gi[:, HID:2 * HID] + gh[:, HID:2 * HID])
        n = jnp.tanh(gi[:, 2 * HID:] + r * gh[:, 2 * HID:])
        xn = (1.0 - z) * n + z * xb
        _msg_out(xn, w0_ref, w1_ref, b0_ref, b1_ref, x_ref_out, ym_ref)

    return pl.pallas_call(
        body,
        grid=(N // BN,),
        in_specs=[pl.BlockSpec((BN, 128), lambda i: (i, 0)),
                  pl.BlockSpec((BN, HID), lambda i: (i, 0)),
                  pl.BlockSpec((HID, 3 * HID), lambda i: (0, 0)),
                  pl.BlockSpec((1, 3 * HID), lambda i: (0, 0)),
                  pl.BlockSpec((HID, 3 * HID), lambda i: (0, 0)),
                  pl.BlockSpec((1, 3 * HID), lambda i: (0, 0))] + _MSG_SPECS,
        out_specs=_MSG_OUT_SPECS,
        out_shape=_MSG_OUT_SHAPES,
        input_output_aliases={1: 0},
    )(agg, x, W_ihT, b_ih2, W_hhT, b_hh2, W_stk0, W_stk1, b_stk0, b_stk1)


def _pool_mlp(x, W1T, b1_2, W2T, b2_2):
    nblk = N // BN

    def body(x_ref, w1_ref, b1_ref, w2_ref, b2_ref, o_ref, acc_ref):
        i = pl.program_id(0)

        @pl.when(i == 0)
        def _():
            acc_ref[...] = jnp.zeros_like(acc_ref)

        acc_ref[...] += jnp.sum(x_ref[...], axis=0, keepdims=True)

        @pl.when(i == nblk - 1)
        def _():
            hg = acc_ref[...] * (1.0 / N)
            t = jnp.maximum(
                jnp.dot(hg, w1_ref[...],
                        preferred_element_type=jnp.float32) + b1_ref[...],
                0.0)
            o_ref[...] = jnp.dot(t, w2_ref[...],
                                 preferred_element_type=jnp.float32) + b2_ref[...]

    return pl.pallas_call(
        body,
        grid=(nblk,),
        in_specs=[pl.BlockSpec((BN, HID), lambda i: (i, 0)),
                  pl.BlockSpec((HID, HID // 2), lambda i: (0, 0)),
                  pl.BlockSpec((1, HID // 2), lambda i: (0, 0)),
                  pl.BlockSpec((HID // 2, NUM_LABELS), lambda i: (0, 0)),
                  pl.BlockSpec((1, NUM_LABELS), lambda i: (0, 0))],
        out_specs=pl.BlockSpec((1, NUM_LABELS), lambda i: (0, 0)),
        out_shape=jax.ShapeDtypeStruct((1, NUM_LABELS), jnp.float32),
        scratch_shapes=[pltpu.VMEM((1, HID), jnp.float32)],
    )(x, W1T, b1_2, W2T, b2_2)


def kernel(h, edge_index, etype, W_in, b_in, W_e, b_e, W_ih, b_ih, W_hh, b_hh,
           W1, b1, W2, b2):
    src = edge_index[0].astype(jnp.int32)
    dst = edge_index[1].astype(jnp.int32)
    et = etype.astype(jnp.int32)

    def gh(hh):
        g = hh * (4 * N) + 4 * src + et
        return jnp.concatenate(
            [g, jnp.zeros((E_PAD - E,), jnp.int32)]).reshape(EDGE_ROWS, LANES)

    g0, g1 = gh(0), gh(1)
    pad_dst = N + jnp.arange(E_PAD - E, dtype=jnp.int32) % (N_PAD - N)
    dst2 = jnp.concatenate([dst, pad_dst]).reshape(EDGE_ROWS, LANES)

    W_inT = W_in.T
    b_in2 = b_in.reshape(1, HID)
    WeT = jnp.transpose(W_e, (0, 2, 1))
    HH = HID // 2
    W_stk0 = jnp.concatenate([WeT[i][:, :HH] for i in range(N_ETYPES)],
                             axis=1)
    W_stk1 = jnp.concatenate([WeT[i][:, HH:] for i in range(N_ETYPES)],
                             axis=1)
    b_stk0 = jnp.concatenate([b_e[i][:HH] for i in range(N_ETYPES)]
                             ).reshape(1, 128)
    b_stk1 = jnp.concatenate([b_e[i][HH:] for i in range(N_ETYPES)]
                             ).reshape(1, 128)
    W_ihT = W_ih.T
    b_ih2 = b_ih.reshape(1, 3 * HID)
    W_hhT = W_hh.T
    b_hh2 = b_hh.reshape(1, 3 * HID)
    W1T = W1.T
    b1_2 = b1.reshape(1, HID // 2)
    W2T = W2.T
    b2_2 = b2.reshape(1, NUM_LABELS)

    x, ym = _lin_msg(h, W_inT, b_in2, W_stk0, W_stk1, b_stk0, b_stk1)

    def step(_, carry):
        xc, ymc = carry
        agg = _sc_scatter_add(ymc.reshape(Y_ROWS, 2 * QW), g0, g1, dst2)
        xn, ymn = _gru_msg(agg, xc, W_ihT, b_ih2, W_hhT, b_hh2,
                           W_stk0, W_stk1, b_stk0, b_stk1)
        return (xn, ymn)

    x, _ym = lax.fori_loop(0, N_STEPS, step, (x, ym))
    return _pool_mlp(x, W1T, b1_2, W2T, b2_2)
